# baseline (speedup 1.0000x reference)
import functools

import jax
import jax.numpy as jnp
from jax import lax
from jax.experimental import pallas as pl
from jax.experimental.pallas import tpu as pltpu
from jax.experimental.pallas import tpu_sc as plsc


def _sc_worker_count():
    try:
        info = plsc.get_sparse_core_info()
        return info.num_cores, info.num_subcores
    except Exception:
        return 2, 16


_IDX_LANES = 128


def _sc_gather_t(table_rows, n_users, users):
    dim = table_rows.shape[0] * _IDX_LANES // n_users
    batch, = users.shape
    nc, ns = _sc_worker_count()
    nw = nc * ns
    b_per_w = batch // nw
    row_q, row_r = divmod(n_users, _IDX_LANES)
    n_pairs = b_per_w // 2
    ring = 4
    assert batch % nw == 0 and b_per_w % 16 == 0 and dim % 16 == 0

    mesh = plsc.VectorSubcoreMesh(core_axis_name="c", subcore_axis_name="s")

    @functools.partial(
        pl.kernel,
        mesh=mesh,
        compiler_params=pltpu.CompilerParams(needs_layout_passes=False),
        out_type=jax.ShapeDtypeStruct((nw, b_per_w, dim), jnp.float32),
        scratch_types=[
            pltpu.VMEM((b_per_w,), jnp.int32),
            pltpu.VMEM((ring, 2 * dim), jnp.int32),
            pltpu.VMEM((ring, 2 * dim, _IDX_LANES), jnp.float32),
            pltpu.VMEM((b_per_w, dim), jnp.float32),
            pltpu.SemaphoreType.DMA((ring,)),
        ],
    )
    def gather_kernel(table_hbm, idx_hbm, out_hbm, idx_v, rows_v, tiles,
                      cols_v, sems):
        wid = lax.axis_index("s") * nc + lax.axis_index("c")
        base = wid * b_per_w
        pltpu.sync_copy(idx_hbm.at[pl.ds(base, b_per_w)], idx_v)

        def u_of(j):
            vec = idx_v[pl.ds((j // 16) * 16, 16)]
            return vec[j % 16]

        iota16 = lax.iota(jnp.int32, 16)

        def fire(p, slot):
            for s in range(2):
                u = u_of(2 * p + s)
                ub = lax.broadcast(u, (16,))
                for d0 in range(0, dim, 16):
                    d_vec = iota16 + d0
                    t = d_vec * row_r + ub
                    rows_v[slot, pl.ds(s * dim + d0, 16)] = (
                        d_vec * row_q + lax.shift_right_logical(t, 7))
            return pltpu.async_copy(
                table_hbm.at[rows_v.at[slot]], tiles.at[slot], sems.at[slot])

        def extract(p, slot):
            for s in range(2):
                u = u_of(2 * p + s)
                ub = lax.broadcast(u, (16,))
                for d0 in range(0, dim, 16):
                    d_vec = iota16 + d0
                    lane = lax.bitwise_and(d_vec * row_r + ub,
                                           lax.broadcast(127, (16,)))
                    rows = iota16 + (s * dim + d0)
                    cols_v[2 * p + s, pl.ds(d0, 16)] = plsc.load_gather(
                        tiles.at[slot], [rows, lane])

        handles = [fire(p, p) for p in range(ring)]
        for p in range(n_pairs):
            slot = p % ring
            handles[slot].wait()
            extract(p, slot)
            if p + ring < n_pairs:
                handles[slot] = fire(p + ring, slot)
        pltpu.sync_copy(cols_v, out_hbm.at[wid])

    return gather_kernel(table_rows, users)


_ITEM_BLK = 4096


def _mm_body(it_ref, ue_ref, out_ref):
    out_ref[...] = lax.dot_general(
        it_ref[...], ue_ref[...],
        (((0,), (1,)), ((), ())),
        preferred_element_type=jnp.float32,
    )


def _tc_scores_t(user_emb, item_t):
    batch, dim = user_emb.shape
    num_items = item_t.shape[1]
    grid = (pl.cdiv(num_items, _ITEM_BLK),)
    return pl.pallas_call(
        _mm_body,
        grid=grid,
        in_specs=[
            pl.BlockSpec((dim, _ITEM_BLK), lambda i: (0, i)),
            pl.BlockSpec((batch, dim), lambda i: (0, 0)),
        ],
        out_specs=pl.BlockSpec((_ITEM_BLK, batch), lambda i: (i, 0)),
        out_shape=jax.ShapeDtypeStruct((num_items, batch), jnp.float32),
        compiler_params=pltpu.CompilerParams(
            dimension_semantics=(pltpu.PARALLEL,),
            vmem_limit_bytes=100 * 1024 * 1024,
        ),
    )(item_t, user_emb)


def kernel(users, user_table, item_table):
    n_users, dim = user_table.shape
    table_rows = user_table.T.reshape(
        dim * n_users // _IDX_LANES, _IDX_LANES)
    slabs = _sc_gather_t(table_rows, n_users, users.astype(jnp.int32))
    user_emb = slabs.reshape(users.shape[0], -1)
    scores_t = _tc_scores_t(user_emb, item_table.T)
    return scores_t.T


# speedup vs baseline: 42.0489x; 42.0489x over previous
import functools

import jax
import jax.numpy as jnp
from jax import lax
from jax.experimental import pallas as pl
from jax.experimental.pallas import tpu as pltpu
from jax.experimental.pallas import tpu_sc as plsc


def _sc_worker_count():
    try:
        info = plsc.get_sparse_core_info()
        return info.num_cores, info.num_subcores
    except Exception:
        return 2, 16


def _sc_gather_t(user_table_t, users):
    dim, _ = user_table_t.shape
    batch, = users.shape
    nc, ns = _sc_worker_count()
    nw = nc * ns
    b_per_w = batch // nw
    wave = 4
    n_waves = b_per_w // wave
    assert batch % nw == 0 and b_per_w % 16 == 0 and dim % 16 == 0

    mesh = plsc.VectorSubcoreMesh(core_axis_name="c", subcore_axis_name="s")

    @functools.partial(
        pl.kernel,
        mesh=mesh,
        compiler_params=pltpu.CompilerParams(needs_layout_passes=False),
        out_type=jax.ShapeDtypeStruct((nw, b_per_w, dim), jnp.float32),
        scratch_types=[
            pltpu.VMEM((b_per_w,), jnp.int32),
            pltpu.VMEM((2, wave, dim, 128), jnp.float32),
            pltpu.VMEM((b_per_w, dim), jnp.float32),
            pltpu.SemaphoreType.DMA((2,)),
        ],
    )
    def gather_kernel(table_hbm, idx_hbm, out_hbm, idx_v, tiles, cols_v, sems):
        wid = lax.axis_index("s") * nc + lax.axis_index("c")
        base = wid * b_per_w
        pltpu.sync_copy(idx_hbm.at[pl.ds(base, b_per_w)], idx_v)

        def u_of(j):
            vec = idx_v[pl.ds((j // 16) * 16, 16)]
            return vec[j % 16]

        def tile_off(u):
            return pl.multiple_of((u // 128) * 128, 128)

        def fire(w, phase):
            handles = []
            for t in range(wave):
                u = u_of(w * wave + t)
                handles.append(pltpu.async_copy(
                    table_hbm.at[:, pl.ds(tile_off(u), 128)],
                    tiles.at[phase, t],
                    sems.at[phase],
                ))
            return handles

        pending = fire(0, 0)
        for w in range(n_waves):
            phase = w % 2
            nxt = fire(w + 1, (w + 1) % 2) if w + 1 < n_waves else []
            for h in pending:
                h.wait()
            pending = nxt
            for t in range(wave):
                j = w * wave + t
                u = u_of(j)
                lane = lax.broadcast(u - tile_off(u), (16,))
                for d0 in range(0, dim, 16):
                    rows = lax.iota(jnp.int32, 16) + d0
                    cols_v[j, pl.ds(d0, 16)] = plsc.load_gather(
                        tiles.at[phase, t], [rows, lane])
        pltpu.sync_copy(cols_v, out_hbm.at[wid])

    return gather_kernel(user_table_t, users)


_ITEM_BLK = 4096


def _mm_body(it_ref, ue_ref, out_ref):
    out_ref[...] = lax.dot_general(
        it_ref[...], ue_ref[...],
        (((0,), (1,)), ((), ())),
        preferred_element_type=jnp.float32,
    )


def _tc_scores_t(user_emb, item_t):
    batch, dim = user_emb.shape
    num_items = item_t.shape[1]
    grid = (pl.cdiv(num_items, _ITEM_BLK),)
    return pl.pallas_call(
        _mm_body,
        grid=grid,
        in_specs=[
            pl.BlockSpec((dim, _ITEM_BLK), lambda i: (0, i)),
            pl.BlockSpec((batch, dim), lambda i: (0, 0)),
        ],
        out_specs=pl.BlockSpec((_ITEM_BLK, batch), lambda i: (i, 0)),
        out_shape=jax.ShapeDtypeStruct((num_items, batch), jnp.float32),
        compiler_params=pltpu.CompilerParams(
            dimension_semantics=(pltpu.PARALLEL,),
            vmem_limit_bytes=100 * 1024 * 1024,
        ),
    )(item_t, user_emb)


def kernel(users, user_table, item_table):
    slabs = _sc_gather_t(user_table.T, users.astype(jnp.int32))
    user_emb = slabs.reshape(users.shape[0], -1)
    scores_t = _tc_scores_t(user_emb, item_table.T)
    return scores_t.T


# speedup vs baseline: 42.2036x; 1.0037x over previous
import functools

import jax
import jax.numpy as jnp
from jax import lax
from jax.experimental import pallas as pl
from jax.experimental.pallas import tpu as pltpu
from jax.experimental.pallas import tpu_sc as plsc


def _sc_worker_count():
    try:
        info = plsc.get_sparse_core_info()
        return info.num_cores, info.num_subcores
    except Exception:
        return 2, 16


def _sc_gather_t(user_table_t, users):
    dim, _ = user_table_t.shape
    batch, = users.shape
    nc, ns = _sc_worker_count()
    nw = nc * ns
    b_per_w = batch // nw
    wave = 4
    n_waves = b_per_w // wave
    assert batch % nw == 0 and b_per_w % 16 == 0 and dim % 16 == 0

    mesh = plsc.VectorSubcoreMesh(core_axis_name="c", subcore_axis_name="s")

    @functools.partial(
        pl.kernel,
        mesh=mesh,
        compiler_params=pltpu.CompilerParams(needs_layout_passes=False),
        out_type=jax.ShapeDtypeStruct((nw, b_per_w, dim), jnp.float32),
        scratch_types=[
            pltpu.VMEM((b_per_w,), jnp.int32),
            pltpu.VMEM((2, wave, dim, 128), jnp.float32),
            pltpu.VMEM((b_per_w, dim), jnp.float32),
            pltpu.SemaphoreType.DMA((2,)),
        ],
    )
    def gather_kernel(table_hbm, idx_hbm, out_hbm, idx_v, tiles, cols_v, sems):
        wid = lax.axis_index("s") * nc + lax.axis_index("c")
        base = wid * b_per_w
        pltpu.sync_copy(idx_hbm.at[pl.ds(base, b_per_w)], idx_v)

        def u_of(j):
            vec = idx_v[pl.ds((j // 16) * 16, 16)]
            return vec[j % 16]

        def tile_off(u):
            return pl.multiple_of((u // 128) * 128, 128)

        def fire(w, phase):
            handles = []
            for t in range(wave):
                u = u_of(w * wave + t)
                handles.append(pltpu.async_copy(
                    table_hbm.at[:, pl.ds(tile_off(u), 128)],
                    tiles.at[phase, t],
                    sems.at[phase],
                ))
            return handles

        pending = fire(0, 0)
        for w in range(n_waves):
            phase = w % 2
            nxt = fire(w + 1, (w + 1) % 2) if w + 1 < n_waves else []
            for h in pending:
                h.wait()
            pending = nxt
            for t in range(wave):
                j = w * wave + t
                u = u_of(j)
                lane = lax.broadcast(u - tile_off(u), (16,))
                for d0 in range(0, dim, 16):
                    rows = lax.iota(jnp.int32, 16) + d0
                    cols_v[j, pl.ds(d0, 16)] = plsc.load_gather(
                        tiles.at[phase, t], [rows, lane])
        pltpu.sync_copy(cols_v, out_hbm.at[wid])

    return gather_kernel(user_table_t, users)


_ITEM_BLK = 6144


def _mm_body(it_ref, ue_ref, out_ref):
    out_ref[...] = lax.dot_general(
        it_ref[...], ue_ref[...],
        (((0,), (1,)), ((), ())),
        preferred_element_type=jnp.float32,
    )


def _tc_scores_t(user_emb, item_t):
    batch, dim = user_emb.shape
    num_items = item_t.shape[1]
    grid = (pl.cdiv(num_items, _ITEM_BLK),)
    return pl.pallas_call(
        _mm_body,
        grid=grid,
        in_specs=[
            pl.BlockSpec((dim, _ITEM_BLK), lambda i: (0, i)),
            pl.BlockSpec((batch, dim), lambda i: (0, 0)),
        ],
        out_specs=pl.BlockSpec((_ITEM_BLK, batch), lambda i: (i, 0)),
        out_shape=jax.ShapeDtypeStruct((num_items, batch), jnp.float32),
        compiler_params=pltpu.CompilerParams(
            dimension_semantics=(pltpu.PARALLEL,),
            vmem_limit_bytes=100 * 1024 * 1024,
        ),
    )(item_t, user_emb)


def kernel(users, user_table, item_table):
    slabs = _sc_gather_t(user_table.T, users.astype(jnp.int32))
    user_emb = slabs.reshape(users.shape[0], -1)
    scores_t = _tc_scores_t(user_emb, item_table.T)
    return scores_t.T
